# baseline (speedup 1.0000x reference)
import jax
import jax.numpy as jnp
from jax import lax
from jax.experimental import pallas as pl
from jax.experimental.pallas import tpu as pltpu
from jax.experimental.pallas import tpu_sc as plsc

NC = 2
NS = 16
LANES = 16
NW = NC * NS
BATCH = 16384
BPW = BATCH // NW
ICHUNK = 128
NCHUNK = BPW // ICHUNK
NF = 16

OFF_W1 = 0
OFF_W2 = OFF_W1 + 64 * 32
OFF_B1 = OFF_W2 + 32 * 64
OFF_B2 = OFF_B1 + 64
OFF_W3 = OFF_B2 + 32
OFF_B3 = OFF_W3 + 32
BLOB = OFF_B3 + 1 + 15


def _body(user_hbm, item_hbm, emb_hbm, blob_hbm, out_hbm,
          idx_u, idx_i, rows_u, rows_i, wv, out_v,
          sem_w, sem_g):
    wid = lax.axis_index("s") * NC + lax.axis_index("c")

    cw = pltpu.async_copy(blob_hbm, wv, sem_w)
    pltpu.sync_copy(user_hbm.at[pl.ds(wid * NCHUNK, NCHUNK)], idx_u)
    pltpu.sync_copy(item_hbm.at[pl.ds(wid * NCHUNK, NCHUNK)], idx_i)

    handles = []
    for j in range(NCHUNK):
        handles.append(pltpu.async_copy(
            emb_hbm.at[idx_u.at[j]],
            rows_u.at[pl.ds(j * ICHUNK, ICHUNK)], sem_g))
    for j in range(NCHUNK):
        handles.append(pltpu.async_copy(
            emb_hbm.at[idx_i.at[j]],
            rows_i.at[pl.ds(j * ICHUNK, ICHUNK)], sem_g))

    cw.wait()
    w3vecs = [wv[pl.ds(OFF_W3 + 16 * c, 16)] for c in range(2)]
    w3s = [w3vecs[j // 16][j % 16] for j in range(32)]
    tvecs = []
    for c4 in range(4):
        acc = jnp.zeros((LANES,), jnp.float32)
        for j in range(32):
            acc = acc + w3s[j] * wv[pl.ds(OFF_W2 + j * 64 + c4 * 16, 16)]
        tvecs.append(acc)
    ts = [tvecs[n // 16][n % 16] for n in range(64)]
    wvecs = []
    for c2 in range(2):
        acc = jnp.zeros((LANES,), jnp.float32)
        for n in range(64):
            acc = acc + ts[n] * wv[pl.ds(OFF_W1 + n * 32 + c2 * 16, 16)]
        wvecs.append(acc)
    s = jnp.zeros((LANES,), jnp.float32)
    for c4 in range(4):
        s = s + tvecs[c4] * wv[pl.ds(OFF_B1 + c4 * 16, 16)]
    for c2 in range(2):
        s = s + w3vecs[c2] * wv[pl.ds(OFF_B2 + c2 * 16, 16)]
    c_eff = jnp.sum(s) + wv[pl.ds(OFF_B3, 16)][0]
    was = [wvecs[0][m] for m in range(NF)]
    wbs = [wvecs[1][m] for m in range(NF)]

    for h in handles:
        h.wait()

    iota = lax.iota(jnp.int32, LANES)
    for g in range(BPW // LANES):
        jvec = iota + (g * LANES)
        accu = jnp.zeros((LANES,), jnp.float32)
        acci = jnp.zeros((LANES,), jnp.float32)
        for k in range(NF):
            kvec = jnp.full((LANES,), k, jnp.int32)
            accu = accu + plsc.load_gather(rows_u, [jvec, kvec]) * was[k]
            acci = acci + plsc.load_gather(rows_i, [jvec, kvec]) * wbs[k]
        acc = accu + acci + c_eff
        out_v[pl.ds(g * LANES, LANES)] = 1.0 / (1.0 + jnp.exp(-acc))

    pltpu.sync_copy(out_v, out_hbm.at[pl.ds(wid * BPW, BPW)])


@jax.jit
def _ncf_sc(user2d, item2d, emb, blob):
    mesh = plsc.VectorSubcoreMesh(core_axis_name="c", subcore_axis_name="s")
    return pl.kernel(
        _body,
        out_type=jax.ShapeDtypeStruct((BATCH,), jnp.float32),
        mesh=mesh,
        compiler_params=pltpu.CompilerParams(
            needs_layout_passes=False, use_tc_tiling_on_sc=False),
        scratch_types=[
            pltpu.VMEM((NCHUNK, ICHUNK), jnp.int32),
            pltpu.VMEM((NCHUNK, ICHUNK), jnp.int32),
            pltpu.VMEM((BPW, NF), jnp.float32),
            pltpu.VMEM((BPW, NF), jnp.float32),
            pltpu.VMEM((BLOB,), jnp.float32),
            pltpu.VMEM((BPW,), jnp.float32),
            pltpu.SemaphoreType.DMA,
            pltpu.SemaphoreType.DMA,
        ],
    )(user2d, item2d, emb, blob)


def kernel(user, item, user_emb, item_emb, W1, b1, W2, b2, W3, b3):
    blob = jnp.concatenate([
        W1.reshape(-1), W2.reshape(-1), b1, b2, W3.reshape(-1), b3,
        jnp.zeros((BLOB - OFF_B3 - 1,), jnp.float32),
    ])
    emb = jnp.concatenate([user_emb, item_emb], axis=0)
    user2d = user.astype(jnp.int32).reshape(NW * NCHUNK, ICHUNK)
    item2d = (item.astype(jnp.int32) + user_emb.shape[0]).reshape(
        NW * NCHUNK, ICHUNK)
    out = _ncf_sc(user2d, item2d, emb, blob)
    return out.reshape(BATCH, 1)


# speedup vs baseline: 1.2090x; 1.2090x over previous
import jax
import jax.numpy as jnp
from jax import lax
from jax.experimental import pallas as pl
from jax.experimental.pallas import tpu as pltpu
from jax.experimental.pallas import tpu_sc as plsc

NC = 2
NS = 16
LANES = 16
NW = NC * NS
BATCH = 16384
BPW = BATCH // NW
ICHUNK = 128
NCHUNK = BPW // ICHUNK
NF = 16

OFF_W1 = 0
OFF_W2 = OFF_W1 + 64 * 32
OFF_B1 = OFF_W2 + 32 * 64
OFF_B2 = OFF_B1 + 64
OFF_W3 = OFF_B2 + 32
OFF_B3 = OFF_W3 + 32
BLOB = OFF_B3 + 1 + 15


def _body(user_hbm, item_hbm, uemb_hbm, iemb_hbm, blob_hbm, out_hbm,
          idx_u, idx_i, rows_u, rows_i, wv, out_v,
          sem_w, sem_g):
    wid = lax.axis_index("s") * NC + lax.axis_index("c")

    cw = pltpu.async_copy(blob_hbm, wv, sem_w)
    pltpu.sync_copy(user_hbm.at[pl.ds(wid * NCHUNK, NCHUNK)], idx_u)
    pltpu.sync_copy(item_hbm.at[pl.ds(wid * NCHUNK, NCHUNK)], idx_i)

    handles = []
    for j in range(NCHUNK):
        handles.append(pltpu.async_copy(
            uemb_hbm.at[idx_u.at[j]],
            rows_u.at[pl.ds(j * ICHUNK, ICHUNK)], sem_g))
    for j in range(NCHUNK):
        handles.append(pltpu.async_copy(
            iemb_hbm.at[idx_i.at[j]],
            rows_i.at[pl.ds(j * ICHUNK, ICHUNK)], sem_g))

    cw.wait()
    w3vecs = [wv[pl.ds(OFF_W3 + 16 * c, 16)] for c in range(2)]
    w3s = [w3vecs[j // 16][j % 16] for j in range(32)]
    tvecs = []
    for c4 in range(4):
        acc = jnp.zeros((LANES,), jnp.float32)
        for j in range(32):
            acc = acc + w3s[j] * wv[pl.ds(OFF_W2 + j * 64 + c4 * 16, 16)]
        tvecs.append(acc)
    ts = [tvecs[n // 16][n % 16] for n in range(64)]
    wvecs = []
    for c2 in range(2):
        acc = jnp.zeros((LANES,), jnp.float32)
        for n in range(64):
            acc = acc + ts[n] * wv[pl.ds(OFF_W1 + n * 32 + c2 * 16, 16)]
        wvecs.append(acc)
    s = jnp.zeros((LANES,), jnp.float32)
    for c4 in range(4):
        s = s + tvecs[c4] * wv[pl.ds(OFF_B1 + c4 * 16, 16)]
    for c2 in range(2):
        s = s + w3vecs[c2] * wv[pl.ds(OFF_B2 + c2 * 16, 16)]
    c_eff = jnp.sum(s) + wv[pl.ds(OFF_B3, 16)][0]
    was = [wvecs[0][m] for m in range(NF)]
    wbs = [wvecs[1][m] for m in range(NF)]

    for h in handles:
        h.wait()

    iota = lax.iota(jnp.int32, LANES)
    for g in range(BPW // LANES):
        jvec = iota + (g * LANES)
        accu = jnp.zeros((LANES,), jnp.float32)
        acci = jnp.zeros((LANES,), jnp.float32)
        for k in range(NF):
            kvec = jnp.full((LANES,), k, jnp.int32)
            accu = accu + plsc.load_gather(rows_u, [jvec, kvec]) * was[k]
            acci = acci + plsc.load_gather(rows_i, [jvec, kvec]) * wbs[k]
        acc = accu + acci + c_eff
        out_v[pl.ds(g * LANES, LANES)] = 1.0 / (1.0 + jnp.exp(-acc))

    pltpu.sync_copy(out_v, out_hbm.at[pl.ds(wid * BPW, BPW)])


@jax.jit
def _ncf_sc(user2d, item2d, user_emb, item_emb, blob):
    mesh = plsc.VectorSubcoreMesh(core_axis_name="c", subcore_axis_name="s")
    return pl.kernel(
        _body,
        out_type=jax.ShapeDtypeStruct((BATCH,), jnp.float32),
        mesh=mesh,
        compiler_params=pltpu.CompilerParams(
            needs_layout_passes=False, use_tc_tiling_on_sc=False),
        scratch_types=[
            pltpu.VMEM((NCHUNK, ICHUNK), jnp.int32),
            pltpu.VMEM((NCHUNK, ICHUNK), jnp.int32),
            pltpu.VMEM((BPW, NF), jnp.float32),
            pltpu.VMEM((BPW, NF), jnp.float32),
            pltpu.VMEM((BLOB,), jnp.float32),
            pltpu.VMEM((BPW,), jnp.float32),
            pltpu.SemaphoreType.DMA,
            pltpu.SemaphoreType.DMA,
        ],
    )(user2d, item2d, user_emb, item_emb, blob)


def kernel(user, item, user_emb, item_emb, W1, b1, W2, b2, W3, b3):
    blob = jnp.concatenate([
        W1.reshape(-1), W2.reshape(-1), b1, b2, W3.reshape(-1), b3,
        jnp.zeros((BLOB - OFF_B3 - 1,), jnp.float32),
    ])
    user2d = user.astype(jnp.int32).reshape(NW * NCHUNK, ICHUNK)
    item2d = item.astype(jnp.int32).reshape(NW * NCHUNK, ICHUNK)
    out = _ncf_sc(user2d, item2d, user_emb, item_emb, blob)
    return out.reshape(BATCH, 1)
